# Initial kernel scaffold; baseline (speedup 1.0000x reference)
import jax
import jax.numpy as jnp
from jax.experimental import pallas as pl


def kernel(doc_codes, tables):
    raise NotImplementedError("write your pallas kernel here")


# speedup vs baseline: 32.0332x; 32.0332x over previous
import functools

import jax
import jax.numpy as jnp
from jax import lax
from jax.experimental import pallas as pl
from jax.experimental.pallas import tpu as pltpu
from jax.experimental.pallas import tpu_sc as plsc

LANES = 16


@functools.partial(jax.jit, static_argnames=("batch", "m", "ksub", "dsub"))
def _pq_gather(codes_flat, table_flat, *, batch, m, ksub, dsub):
    info = plsc.get_sparse_core_info()
    nc, ns = info.num_cores, info.num_subcores
    nw = nc * ns
    total = batch * m
    per_w = total // nw
    sbi = 6144
    nsb = per_w // sbi
    nvec = sbi // LANES
    groups = m // LANES

    mesh = plsc.VectorSubcoreMesh(core_axis_name="c", subcore_axis_name="s")

    @functools.partial(
        pl.kernel,
        mesh=mesh,
        compiler_params=pltpu.CompilerParams(use_tc_tiling_on_sc=False),
        out_type=jax.ShapeDtypeStruct((total, dsub), jnp.float32),
        scratch_types=[
            pltpu.VMEM((sbi,), jnp.int32),
            pltpu.VMEM((sbi, dsub), jnp.float32),
            pltpu.SemaphoreType.DMA,
        ],
    )
    def k(codes_hbm, table_hbm, out_hbm, idx_v, data_v, sem):
        wid = lax.axis_index("s") * nc + lax.axis_index("c")
        base = wid * per_w

        def sb_body(sb, carry):
            off = base + sb * sbi
            pltpu.sync_copy(codes_hbm.at[pl.ds(off, sbi)], idx_v)

            def add_body(v, carry2):
                g = lax.rem(v, groups)
                offv = lax.iota(jnp.int32, LANES) * ksub + g * (LANES * ksub)
                idx_v[pl.ds(v * LANES, LANES)] = (
                    idx_v[pl.ds(v * LANES, LANES)] + offv
                )
                return carry2

            lax.fori_loop(0, nvec, add_body, 0)
            pltpu.async_copy(table_hbm.at[idx_v], data_v, sem).wait()
            pltpu.sync_copy(data_v, out_hbm.at[pl.ds(off, sbi)])
            return carry

        lax.fori_loop(0, nsb, sb_body, 0)

    return k(codes_flat, table_flat)


def kernel(doc_codes, tables):
    batch, m = doc_codes.shape
    _, ksub, dsub = tables.shape
    codes_flat = doc_codes.reshape(-1).astype(jnp.int32)
    table_flat = tables.reshape(m * ksub, dsub)
    out = _pq_gather(codes_flat, table_flat, batch=batch, m=m, ksub=ksub, dsub=dsub)
    return out.reshape(batch, m * dsub)


# speedup vs baseline: 36.0610x; 1.1257x over previous
import functools

import jax
import jax.numpy as jnp
from jax import lax
from jax.experimental import pallas as pl
from jax.experimental.pallas import tpu as pltpu
from jax.experimental.pallas import tpu_sc as plsc

LANES = 16


@functools.partial(jax.jit, static_argnames=("batch", "m", "ksub", "dsub"))
def _pq_gather(codes_flat, table_flat, *, batch, m, ksub, dsub):
    info = plsc.get_sparse_core_info()
    nc, ns = info.num_cores, info.num_subcores
    nw = nc * ns
    total = batch * m
    per_w = total // nw
    sbi = 6144
    nsb = per_w // sbi
    groups = m // LANES
    rows_per_sb = sbi // m

    mesh = plsc.VectorSubcoreMesh(core_axis_name="c", subcore_axis_name="s")

    @functools.partial(
        pl.kernel,
        mesh=mesh,
        compiler_params=pltpu.CompilerParams(use_tc_tiling_on_sc=False),
        out_type=jax.ShapeDtypeStruct((total, dsub), jnp.float32),
        scratch_types=[
            pltpu.VMEM((sbi,), jnp.int32),
            pltpu.VMEM((sbi,), jnp.int32),
            pltpu.VMEM((sbi, dsub), jnp.float32),
            pltpu.VMEM((sbi, dsub), jnp.float32),
            pltpu.SemaphoreType.DMA,
            pltpu.SemaphoreType.DMA,
            pltpu.SemaphoreType.DMA,
            pltpu.SemaphoreType.DMA,
        ],
    )
    def k(codes_hbm, table_hbm, out_hbm, idx0, idx1, dat0, dat1,
          gsem0, gsem1, wsem0, wsem1):
        wid = lax.axis_index("s") * nc + lax.axis_index("c")
        base = wid * per_w
        idx = (idx0, idx1)
        dat = (dat0, dat1)
        gsem = (gsem0, gsem1)
        wsem = (wsem0, wsem1)

        def prep(sb, b):
            off = base + sb * sbi
            pltpu.sync_copy(codes_hbm.at[pl.ds(off, sbi)], idx[b])

            def row_body(r, carry):
                for g in range(groups):
                    offv = lax.iota(jnp.int32, LANES) * ksub + g * (LANES * ksub)
                    s = r * m + g * LANES
                    idx[b][pl.ds(s, LANES)] = idx[b][pl.ds(s, LANES)] + offv
                return carry

            lax.fori_loop(0, rows_per_sb, row_body, 0)

        def gather_start(sb, b):
            pltpu.async_copy(table_hbm.at[idx[b]], dat[b], gsem[b])

        def gather_wait(b):
            pltpu.make_async_copy(table_hbm.at[idx[b]], dat[b], gsem[b]).wait()

        def write_start(sb, b):
            off = base + sb * sbi
            pltpu.async_copy(dat[b], out_hbm.at[pl.ds(off, sbi)], wsem[b])

        def write_wait(sb, b):
            off = base + sb * sbi
            pltpu.make_async_copy(dat[b], out_hbm.at[pl.ds(off, sbi)], wsem[b]).wait()

        prep(0, 0)
        gather_start(0, 0)

        def step(cur, b):
            nxt = cur + 1

            @pl.when(nxt < nsb)
            def _():
                prep(nxt, 1 - b)

            @pl.when(cur >= 1)
            def _():
                write_wait(cur - 1, 1 - b)

            @pl.when(nxt < nsb)
            def _():
                gather_start(nxt, 1 - b)

            gather_wait(b)
            write_start(cur, b)

        def pair_body(s, carry):
            step(s * 2, 0)
            step(s * 2 + 1, 1)
            return carry

        lax.fori_loop(0, nsb // 2, pair_body, 0)
        write_wait(nsb - 1, 1)

    return k(codes_flat, table_flat)


def kernel(doc_codes, tables):
    batch, m = doc_codes.shape
    _, ksub, dsub = tables.shape
    codes_flat = doc_codes.reshape(-1).astype(jnp.int32)
    table_flat = tables.reshape(m * ksub, dsub)
    out = _pq_gather(codes_flat, table_flat, batch=batch, m=m, ksub=ksub, dsub=dsub)
    return out.reshape(batch, m * dsub)


# speedup vs baseline: 86.3629x; 2.3949x over previous
import functools

import jax
import jax.numpy as jnp
from jax import lax
from jax.experimental import pallas as pl
from jax.experimental.pallas import tpu as pltpu
from jax.experimental.pallas import tpu_sc as plsc

LANES = 16


@functools.partial(jax.jit, static_argnames=("batch", "m", "ksub", "dsub"))
def _pq_gather(codes_x, table_y, *, batch, m, ksub, dsub):
    info = plsc.get_sparse_core_info()
    nc, ns = info.num_cores, info.num_subcores
    nw = nc * ns
    total = batch * m
    per_w = total // nw
    sbi = 6144
    nsb = per_w // sbi
    groups = m // LANES
    rows_per_sb = sbi // m
    kc = ksub // 128
    m_sc = m // ns

    mesh = plsc.VectorSubcoreMesh(core_axis_name="c", subcore_axis_name="s")

    @functools.partial(
        pl.kernel,
        mesh=mesh,
        compiler_params=pltpu.CompilerParams(use_tc_tiling_on_sc=False),
        out_type=jax.ShapeDtypeStruct((total, dsub), jnp.float32),
        scratch_types=[
            pltpu.VMEM((sbi,), jnp.int32),
            pltpu.VMEM((sbi,), jnp.int32),
            pltpu.VMEM((rows_per_sb, 128), jnp.int32),
            pltpu.VMEM((sbi, dsub), jnp.float32),
            pltpu.VMEM((sbi, dsub), jnp.float32),
            pltpu.VMEM_SHARED((m * ksub, dsub), jnp.float32),
            pltpu.SemaphoreType.DMA,
            pltpu.SemaphoreType.DMA,
            pltpu.SemaphoreType.DMA,
            pltpu.SemaphoreType.DMA,
        ],
    )
    def k(codes_hbm, table_hbm, out_hbm, idx0, idx1, cbuf, dat0, dat1,
          table_sh, gsem0, gsem1, wsem0, wsem1):
        cid = lax.axis_index("c")
        sid = lax.axis_index("s")
        wid = sid * nc + cid
        base = wid * per_w

        iota = lax.iota(jnp.int32, LANES)
        iota_div8 = lax.shift_right_logical(iota, 3)
        iota_mod8 = lax.bitwise_and(iota, 7)

        @pl.when(sid == 0)
        def _():
            pltpu.sync_copy(table_hbm, table_sh)

        plsc.subcore_barrier()

        idx = (idx0, idx1)
        dat = (dat0, dat1)
        gsem = (gsem0, gsem1)
        wsem = (wsem0, wsem1)

        def prep(sb, b):
            row0 = (base + sb * sbi) // m
            pltpu.sync_copy(codes_hbm.at[pl.ds(row0, rows_per_sb), :], cbuf)

            def row_body(r, carry):
                rhi = lax.shift_right_logical(r, 3)
                rlo = lax.bitwise_and(r, 7)
                rsplat = jnp.full((LANES,), r, jnp.int32)
                for g in range(groups):
                    offv = iota * ksub + g * (LANES * ksub)
                    codes_vec = cbuf[r, pl.ds(g * LANES, LANES)]
                    t = (rhi * groups + g) * 8 + rlo
                    idx[b][pl.ds(t * LANES, LANES)] = codes_vec + offv
                return carry

            lax.fori_loop(0, rows_per_sb, row_body, 0)

        def gather_start(sb, b):
            pltpu.async_copy(table_sh.at[idx[b]], dat[b], gsem[b])

        def gather_wait(b):
            pltpu.make_async_copy(table_sh.at[idx[b]], dat[b], gsem[b]).wait()

        def write_start(sb, b):
            off = base + sb * sbi
            pltpu.async_copy(dat[b], out_hbm.at[pl.ds(off, sbi)], wsem[b])

        def write_wait(sb, b):
            off = base + sb * sbi
            pltpu.make_async_copy(dat[b], out_hbm.at[pl.ds(off, sbi)], wsem[b]).wait()

        prep(0, 0)
        gather_start(0, 0)

        def step(cur, b):
            nxt = cur + 1

            @pl.when(nxt < nsb)
            def _():
                prep(nxt, 1 - b)

            @pl.when(cur >= 1)
            def _():
                write_wait(cur - 1, 1 - b)

            @pl.when(nxt < nsb)
            def _():
                gather_start(nxt, 1 - b)

            gather_wait(b)
            write_start(cur, b)

        def pair_body(s, carry):
            step(s * 2, 0)
            step(s * 2 + 1, 1)
            return carry

        lax.fori_loop(0, nsb // 2, pair_body, 0)
        write_wait(nsb - 1, 1)

    return k(codes_x, table_y)


def kernel(doc_codes, tables):
    batch, m = doc_codes.shape
    _, ksub, dsub = tables.shape
    codes = doc_codes.astype(jnp.int32)
    codes_x = jnp.pad(codes, ((0, 0), (0, 128 - m)))
    table_flat = tables.reshape(m * ksub, dsub)
    out = _pq_gather(codes_x, table_flat, batch=batch, m=m, ksub=ksub, dsub=dsub)
    ngrp = m * dsub // 128
    out = out.reshape(batch // 8, ngrp, 8, 128).transpose(0, 2, 1, 3)
    return out.reshape(batch, m * dsub)


# speedup vs baseline: 88.2150x; 1.0214x over previous
import functools

import jax
import jax.numpy as jnp
from jax import lax
from jax.experimental import pallas as pl
from jax.experimental.pallas import tpu as pltpu
from jax.experimental.pallas import tpu_sc as plsc

LANES = 16


@functools.partial(jax.jit, static_argnames=("batch", "m", "ksub", "dsub"))
def _pq_gather(codes_x, table_y, *, batch, m, ksub, dsub):
    info = plsc.get_sparse_core_info()
    nc, ns = info.num_cores, info.num_subcores
    nw = nc * ns
    total = batch * m
    per_w = total // nw
    rows_per_sb = 32
    sbi = rows_per_sb * m
    nsb = per_w // sbi
    groups = m // LANES
    rtiles = m // 8
    kc = ksub // 128
    m_sc = m // ns

    mesh = plsc.VectorSubcoreMesh(core_axis_name="c", subcore_axis_name="s")

    @functools.partial(
        pl.kernel,
        mesh=mesh,
        compiler_params=pltpu.CompilerParams(
            use_tc_tiling_on_sc=False, needs_layout_passes=False
        ),
        out_type=jax.ShapeDtypeStruct((total, dsub), jnp.float32),
        scratch_types=[
            pltpu.VMEM((sbi,), jnp.int32),
            pltpu.VMEM((sbi,), jnp.int32),
            pltpu.VMEM((rtiles, 8, 128), jnp.int32),
            pltpu.VMEM((sbi, dsub), jnp.float32),
            pltpu.VMEM((sbi, dsub), jnp.float32),
            pltpu.VMEM((m_sc, kc, dsub, 128), jnp.float32),
            pltpu.VMEM((m_sc * ksub, dsub), jnp.float32),
            pltpu.VMEM_SHARED((m * ksub, dsub), jnp.float32),
            pltpu.SemaphoreType.DMA,
            pltpu.SemaphoreType.DMA,
            pltpu.SemaphoreType.DMA,
            pltpu.SemaphoreType.DMA,
            pltpu.SemaphoreType.DMA,
        ],
    )
    def k(codes_hbm, table_hbm, out_hbm, idx0, idx1, cbuf, dat0, dat1,
          ybuf, trows, table_sh, gsem0, gsem1, wsem0, wsem1, csem):
        cid = lax.axis_index("c")
        sid = lax.axis_index("s")
        wid = sid * nc + cid
        base = wid * per_w

        iota = lax.iota(jnp.int32, LANES)
        iota_div8 = lax.shift_right_logical(iota, 3)
        iota_mod8 = lax.bitwise_and(iota, 7)

        pltpu.sync_copy(table_hbm.at[pl.ds(sid * m_sc, m_sc)], ybuf)

        def stage_j(j, carry):
            for cc in range(kc):
                for d in range(dsub):
                    for w in range(128 // LANES):
                        vals = ybuf[j, cc, d, pl.ds(w * LANES, LANES)]
                        i0 = iota + (j * ksub + cc * 128 + w * LANES)
                        i1 = jnp.full((LANES,), d, jnp.int32)
                        plsc.store_scatter(trows, [i0, i1], vals)
            return carry

        lax.fori_loop(0, m_sc, stage_j, 0)
        pltpu.sync_copy(
            trows, table_sh.at[pl.ds(sid * (m_sc * ksub), m_sc * ksub), :]
        )
        plsc.subcore_barrier()

        idx = (idx0, idx1)
        dat = (dat0, dat1)
        gsem = (gsem0, gsem1)
        wsem = (wsem0, wsem1)

        def prep(sb, b):
            row0 = (base + sb * sbi) // m
            c_tile = row0 // 128
            c_off = row0 % 128

            @pl.when(c_off == 0)
            def _():
                for rt in range(rtiles):
                    pltpu.async_copy(
                        codes_hbm.at[rt, c_tile], cbuf.at[rt], csem
                    )
                for rt in range(rtiles):
                    pltpu.make_async_copy(
                        codes_hbm.at[rt, c_tile], cbuf.at[rt], csem
                    ).wait()

            def row_body(r, carry):
                rhi = lax.shift_right_logical(r, 3)
                rlo = lax.bitwise_and(r, 7)
                csplat = jnp.full((LANES,), c_off + r, jnp.int32)
                for g in range(groups):
                    i0 = iota_div8 + (2 * g)
                    codes_vec = plsc.load_gather(
                        cbuf, [i0, iota_mod8, csplat]
                    )
                    offv = iota * ksub + g * (LANES * ksub)
                    t = (rhi * groups + g) * 8 + rlo
                    idx[b][pl.ds(t * LANES, LANES)] = codes_vec + offv
                return carry

            lax.fori_loop(0, rows_per_sb, row_body, 0)

        def gather_start(sb, b):
            pltpu.async_copy(table_sh.at[idx[b]], dat[b], gsem[b])

        def gather_wait(b):
            pltpu.make_async_copy(table_sh.at[idx[b]], dat[b], gsem[b]).wait()

        def write_start(sb, b):
            off = base + sb * sbi
            pltpu.async_copy(dat[b], out_hbm.at[pl.ds(off, sbi)], wsem[b])

        def write_wait(sb, b):
            off = base + sb * sbi
            pltpu.make_async_copy(dat[b], out_hbm.at[pl.ds(off, sbi)], wsem[b]).wait()

        prep(0, 0)
        gather_start(0, 0)

        def step(cur, b):
            nxt = cur + 1

            @pl.when(nxt < nsb)
            def _():
                prep(nxt, 1 - b)

            @pl.when(cur >= 1)
            def _():
                write_wait(cur - 1, 1 - b)

            @pl.when(nxt < nsb)
            def _():
                gather_start(nxt, 1 - b)

            gather_wait(b)
            write_start(cur, b)

        def pair_body(s, carry):
            step(s * 2, 0)
            step(s * 2 + 1, 1)
            return carry

        lax.fori_loop(0, nsb // 2, pair_body, 0)
        write_wait(nsb - 1, 1)

    return k(codes_x, table_y)


def kernel(doc_codes, tables):
    batch, m = doc_codes.shape
    _, ksub, dsub = tables.shape
    codes = doc_codes.astype(jnp.int32)
    codes_x = (
        codes.T.reshape(m // 8, 8, batch // 128, 128).transpose(0, 2, 1, 3)
    )
    table_y = tables.reshape(m, ksub // 128, 128, dsub).transpose(0, 1, 3, 2)
    out = _pq_gather(codes_x, table_y, batch=batch, m=m, ksub=ksub, dsub=dsub)
    ngrp = m * dsub // 128
    out = out.reshape(batch // 8, ngrp, 8, 128).transpose(0, 2, 1, 3)
    return out.reshape(batch, m * dsub)


# speedup vs baseline: 91.6816x; 1.0393x over previous
import functools

import jax
import jax.numpy as jnp
from jax import lax
from jax.experimental import pallas as pl
from jax.experimental.pallas import tpu as pltpu
from jax.experimental.pallas import tpu_sc as plsc

LANES = 16


@functools.partial(jax.jit, static_argnames=("batch", "m", "ksub", "dsub"))
def _pq_gather(codes_x, table_y, *, batch, m, ksub, dsub):
    info = plsc.get_sparse_core_info()
    nc, ns = info.num_cores, info.num_subcores
    nw = nc * ns
    total = batch * m
    per_w = total // nw
    rows_per_sb = 32
    sbi = rows_per_sb * m
    nsb = per_w // sbi
    groups = m // LANES
    rtiles = m // 8
    kc = ksub // 128
    m_sc = m // ns

    mesh = plsc.VectorSubcoreMesh(core_axis_name="c", subcore_axis_name="s")

    @functools.partial(
        pl.kernel,
        mesh=mesh,
        compiler_params=pltpu.CompilerParams(
            use_tc_tiling_on_sc=False, needs_layout_passes=False
        ),
        out_type=jax.ShapeDtypeStruct((total, dsub), jnp.float32),
        scratch_types=[
            pltpu.VMEM((sbi,), jnp.int32),
            pltpu.VMEM((sbi,), jnp.int32),
            pltpu.VMEM((rtiles, 8, 128), jnp.int32),
            pltpu.VMEM((rtiles, 8, 128), jnp.int32),
            pltpu.VMEM((sbi, dsub), jnp.float32),
            pltpu.VMEM((sbi, dsub), jnp.float32),
            pltpu.VMEM((m_sc, kc, dsub, 128), jnp.float32),
            pltpu.VMEM((m_sc * ksub, dsub), jnp.float32),
            pltpu.VMEM_SHARED((m * ksub, dsub), jnp.float32),
            pltpu.SemaphoreType.DMA,
            pltpu.SemaphoreType.DMA,
            pltpu.SemaphoreType.DMA,
            pltpu.SemaphoreType.DMA,
            pltpu.SemaphoreType.DMA,
        ],
    )
    def k(codes_hbm, table_hbm, out_hbm, idx0, idx1, cbufa, cbufb, dat0,
          dat1, ybuf, trows, table_sh, gsem0, gsem1, wsem0, wsem1, csem):
        cid = lax.axis_index("c")
        sid = lax.axis_index("s")
        wid = sid * nc + cid
        base = wid * per_w

        iota = lax.iota(jnp.int32, LANES)
        iota_div8 = lax.shift_right_logical(iota, 3)
        iota_mod8 = lax.bitwise_and(iota, 7)

        cbufs = (cbufa, cbufb)
        base_tile = wid * (per_w // m // 128)

        def fetch_start(tc, buf):
            for rt in range(rtiles):
                pltpu.async_copy(
                    codes_hbm.at[rt, base_tile + tc], buf.at[rt], csem
                )

        def fetch_wait(tc, buf):
            for rt in range(rtiles):
                pltpu.make_async_copy(
                    codes_hbm.at[rt, base_tile + tc], buf.at[rt], csem
                ).wait()

        fetch_start(0, cbufs[0])

        pltpu.sync_copy(table_hbm.at[pl.ds(sid * m_sc, m_sc)], ybuf)

        def stage_j(j, carry):
            for cc in range(kc):
                for d in range(dsub):
                    for w in range(128 // LANES):
                        vals = ybuf[j, cc, d, pl.ds(w * LANES, LANES)]
                        i0 = iota + (j * ksub + cc * 128 + w * LANES)
                        i1 = jnp.full((LANES,), d, jnp.int32)
                        plsc.store_scatter(trows, [i0, i1], vals)
            return carry

        lax.fori_loop(0, m_sc, stage_j, 0)
        pltpu.sync_copy(
            trows, table_sh.at[pl.ds(sid * (m_sc * ksub), m_sc * ksub), :]
        )
        plsc.subcore_barrier()

        idx = (idx0, idx1)
        dat = (dat0, dat1)
        gsem = (gsem0, gsem1)
        wsem = (wsem0, wsem1)

        def prep(sb, b, buf):
            c_off = (sb % 4) * rows_per_sb

            def row_body(r, carry):
                rhi = lax.shift_right_logical(r, 3)
                rlo = lax.bitwise_and(r, 7)
                csplat = jnp.full((LANES,), c_off + r, jnp.int32)
                for g in range(groups):
                    i0 = iota_div8 + (2 * g)
                    codes_vec = plsc.load_gather(
                        buf, [i0, iota_mod8, csplat]
                    )
                    offv = iota * ksub + g * (LANES * ksub)
                    t = (rhi * groups + g) * 8 + rlo
                    idx[b][pl.ds(t * LANES, LANES)] = codes_vec + offv
                return carry

            lax.fori_loop(0, rows_per_sb, row_body, 0)

        def gather_start(sb, b):
            pltpu.async_copy(table_sh.at[idx[b]], dat[b], gsem[b])

        def gather_wait(b):
            pltpu.make_async_copy(table_sh.at[idx[b]], dat[b], gsem[b]).wait()

        def write_start(sb, b):
            off = base + sb * sbi
            pltpu.async_copy(dat[b], out_hbm.at[pl.ds(off, sbi)], wsem[b])

        def write_wait(sb, b):
            off = base + sb * sbi
            pltpu.make_async_copy(dat[b], out_hbm.at[pl.ds(off, sbi)], wsem[b]).wait()

        ntiles = nsb // 4
        fetch_wait(0, cbufs[0])
        prep(0, 0, cbufs[0])
        gather_start(0, 0)
        for tc in range(ntiles):
            if tc + 1 < ntiles:
                fetch_start(tc + 1, cbufs[(tc + 1) % 2])
            for ss in range(4):
                cur = tc * 4 + ss
                b = cur % 2
                nxt = cur + 1
                if nxt < nsb:
                    nbuf = cbufs[(nxt // 4) % 2]
                    if nxt % 4 == 0:
                        fetch_wait(nxt // 4, nbuf)
                    prep(nxt, 1 - b, nbuf)
                if cur >= 1:
                    write_wait(cur - 1, 1 - b)
                if nxt < nsb:
                    gather_start(nxt, 1 - b)
                gather_wait(b)
                write_start(cur, b)
        write_wait(nsb - 1, 1)

    return k(codes_x, table_y)


def kernel(doc_codes, tables):
    batch, m = doc_codes.shape
    _, ksub, dsub = tables.shape
    codes = doc_codes.astype(jnp.int32)
    codes_x = (
        codes.T.reshape(m // 8, 8, batch // 128, 128).transpose(0, 2, 1, 3)
    )
    table_y = tables.reshape(m, ksub // 128, 128, dsub).transpose(0, 1, 3, 2)
    out = _pq_gather(codes_x, table_y, batch=batch, m=m, ksub=ksub, dsub=dsub)
    ngrp = m * dsub // 128
    out = out.reshape(batch // 8, ngrp, 8, 128).transpose(0, 2, 1, 3)
    return out.reshape(batch, m * dsub)
